# Initial kernel scaffold; baseline (speedup 1.0000x reference)
import jax
import jax.numpy as jnp
from jax.experimental import pallas as pl


def kernel(h, edge_index, W1a, b1a, W1b, b1b, W2a, b2a, W2b, b2b, W3a, b3a, W3b, b3b, W4a, b4a, W4b, b4b, W5a, b5a, W5b, b5b, l1, l2, l3, l4, l5):
    raise NotImplementedError("write your pallas kernel here")


# speedup vs baseline: 2.4989x; 2.4989x over previous
import functools

import jax
import jax.numpy as jnp
from jax import lax
from jax.experimental import pallas as pl
from jax.experimental.pallas import tpu as pltpu
from jax.experimental.pallas import tpu_sc as plsc

N = 10000
CH = 128
EB = 128
NS = 16
NCORES = 2
ACC_ROWS = 10240
ZPT = ACC_ROWS // NS
DUMMY_ROW = N


def _pad_edges(e):
    bpt = -(-e // (NS * EB))
    return bpt, NS * bpt * EB


def _segsum(x_flat, srcs_flat, dst_e, zrows, nc, bpt, ep):
    rounds = nc // NCORES
    mesh = plsc.VectorSubcoreMesh(
        core_axis_name="c", subcore_axis_name="s",
        num_cores=NCORES, num_subcores=NS)

    @functools.partial(
        pl.kernel,
        out_type=jax.ShapeDtypeStruct((nc * ACC_ROWS, CH), jnp.float32),
        mesh=mesh,
        scratch_types=[
            pltpu.VMEM((EB,), jnp.int32),
            pltpu.VMEM((EB,), jnp.int32),
            pltpu.VMEM((EB, CH), jnp.float32),
            pltpu.VMEM_SHARED((ACC_ROWS, CH), jnp.float32),
            pltpu.SemaphoreType.DMA,
        ],
    )
    def k(x_hbm, srcs_hbm, dst_hbm, z_hbm, out_hbm, sidx, didx, rows, acc, sem):
        c = lax.axis_index("c")
        s = lax.axis_index("s")
        for r in range(rounds):
            chunk = c * rounds + r
            pltpu.sync_copy(z_hbm, acc.at[pl.ds(s * ZPT, ZPT)])
            plsc.subcore_barrier()

            def eblock(b, _):
                ebase = (s * bpt + b) * EB
                pltpu.sync_copy(srcs_hbm.at[pl.ds(chunk * ep + ebase, EB)], sidx)
                pltpu.sync_copy(dst_hbm.at[pl.ds(ebase, EB)], didx)
                pltpu.async_copy(x_hbm.at[sidx], rows, sem).wait()
                pltpu.sync_copy(rows, acc.at[didx], add=True)
                return 0

            lax.fori_loop(0, bpt, eblock, 0)
            plsc.subcore_barrier()
            pltpu.sync_copy(acc.at[pl.ds(s * ZPT, ZPT)],
                            out_hbm.at[pl.ds(chunk * ACC_ROWS + s * ZPT, ZPT)])
            plsc.subcore_barrier()

    return k(x_flat, srcs_flat, dst_e, zrows)


def _mlp(x_ch, agg_ch, wa, ba, wb, bb):
    nc, n, _ = x_ch.shape
    dim = wb.shape[1]
    nco = dim // CH
    rb = 1000
    grid = (n // rb,)

    def body(x_ref, agg_ref, wa_ref, ba_ref, wb_ref, bb_ref, out_ref, cs_ref):
        i = pl.program_id(0)
        acc = jnp.zeros((rb, dim), jnp.float32)
        for c in range(nc):
            z = x_ref[c] + agg_ref[c]
            acc = acc + jnp.dot(z, wa_ref[c * CH:(c + 1) * CH, :],
                                preferred_element_type=jnp.float32)
        h1 = jnp.maximum(acc + ba_ref[...], 0.0)
        out = jnp.dot(h1, wb_ref[...], preferred_element_type=jnp.float32)
        out = jnp.maximum(out + bb_ref[...], 0.0)
        for c in range(nco):
            out_ref[c] = out[:, c * CH:(c + 1) * CH]

        @pl.when(i == 0)
        def _():
            cs_ref[...] = jnp.zeros_like(cs_ref)

        cs_ref[...] += jnp.sum(out, axis=0, keepdims=True)

    return pl.pallas_call(
        body,
        grid=grid,
        in_specs=[
            pl.BlockSpec((nc, rb, CH), lambda i: (0, i, 0)),
            pl.BlockSpec((nc, rb, CH), lambda i: (0, i, 0)),
            pl.BlockSpec(wa.shape, lambda i: (0, 0)),
            pl.BlockSpec(ba.shape, lambda i: (0, 0)),
            pl.BlockSpec(wb.shape, lambda i: (0, 0)),
            pl.BlockSpec(bb.shape, lambda i: (0, 0)),
        ],
        out_specs=[
            pl.BlockSpec((nco, rb, CH), lambda i: (0, i, 0)),
            pl.BlockSpec((1, dim), lambda i: (0, 0)),
        ],
        out_shape=[
            jax.ShapeDtypeStruct((nco, n, CH), jnp.float32),
            jax.ShapeDtypeStruct((1, dim), jnp.float32),
        ],
        compiler_params=pltpu.CompilerParams(
            dimension_semantics=("arbitrary",)),
    )(x_ch, agg_ch, wa, ba, wb, bb)


def _readout(cs_list, lt_list):
    def body(*refs):
        out_ref = refs[-1]
        tot = jnp.zeros((1, 1), jnp.float32)
        for i in range(5):
            cs = refs[i][...]
            lt = refs[5 + i][...]
            tot = tot + jnp.sum(cs * lt, axis=1, keepdims=True)
        tot = tot * (1.0 / N)
        out_ref[...] = 1.0 / (1.0 + jnp.exp(-tot))

    return pl.pallas_call(
        body,
        out_shape=jax.ShapeDtypeStruct((1, 1), jnp.float32),
    )(*cs_list, *lt_list)


def kernel(h, edge_index, W1a, b1a, W1b, b1b, W2a, b2a, W2b, b2b, W3a, b3a,
           W3b, b3b, W4a, b4a, W4b, b4b, W5a, b5a, W5b, b5b, l1, l2, l3, l4,
           l5):
    n, f_in = h.shape
    e = edge_index.shape[1]
    src = edge_index[0].astype(jnp.int32)
    dst = edge_index[1].astype(jnp.int32)
    bpt, ep = _pad_edges(e)
    pad = ep - e
    src_p = jnp.concatenate([src, jnp.zeros((pad,), jnp.int32)])
    dst_p = jnp.concatenate([dst, jnp.full((pad,), DUMMY_ROW, jnp.int32)])
    srcs = {
        ncv: jnp.concatenate([src_p + c * N for c in range(ncv)])
        for ncv in (f_in // CH, 4)
    }
    zrows = jnp.zeros((ZPT, CH), jnp.float32)

    params = [(W1a, b1a, W1b, b1b), (W2a, b2a, W2b, b2b),
              (W3a, b3a, W3b, b3b), (W4a, b4a, W4b, b4b),
              (W5a, b5a, W5b, b5b)]

    x = h.reshape(n, f_in // CH, CH).transpose(1, 0, 2)
    cs_list = []
    for wa, ba, wb, bb in params:
        nc = x.shape[0]
        agg_flat = _segsum(x.reshape(nc * N, CH), srcs[nc], dst_p, zrows,
                           nc, bpt, ep)
        agg = agg_flat.reshape(nc, ACC_ROWS, CH)
        x, cs = _mlp(x, agg, wa, ba.reshape(1, -1), wb, bb.reshape(1, -1))
        cs_list.append(cs)

    lt_list = [l.reshape(1, -1) for l in (l1, l2, l3, l4, l5)]
    return _readout(cs_list, lt_list)


# speedup vs baseline: 4.0205x; 1.6089x over previous
import functools

import jax
import jax.numpy as jnp
from jax import lax
from jax.experimental import pallas as pl
from jax.experimental.pallas import tpu as pltpu
from jax.experimental.pallas import tpu_sc as plsc

N = 10000
CH = 128
EB = 128
NS = 16
NCORES = 2
ACC_ROWS = 10240
ZPT = ACC_ROWS // NS
DUMMY_ROW = N


def _pad_edges(e):
    bpt = -(-e // (NS * EB))
    return bpt, NS * bpt * EB


def _segsum(x_flat, srcs_flat, dst_e, zrows, nc, bpt, ep):
    rounds = nc // NCORES
    nslot = 2
    mesh = plsc.VectorSubcoreMesh(
        core_axis_name="c", subcore_axis_name="s",
        num_cores=NCORES, num_subcores=NS)

    scratch = [pltpu.VMEM((bpt * EB,), jnp.int32)]
    scratch += [pltpu.VMEM((EB,), jnp.int32) for _ in range(nslot)]
    scratch += [pltpu.VMEM((EB, CH), jnp.float32) for _ in range(nslot)]
    scratch += [pltpu.VMEM_SHARED((ACC_ROWS, CH), jnp.float32)]
    scratch += [pltpu.SemaphoreType.DMA for _ in range(2 * nslot)]

    @functools.partial(
        pl.kernel,
        out_type=jax.ShapeDtypeStruct((nc * ACC_ROWS, CH), jnp.float32),
        mesh=mesh,
        scratch_types=scratch,
    )
    def k(x_hbm, srcs_hbm, dst_hbm, z_hbm, out_hbm, sall, *rest):
        didx = rest[0:nslot]
        rows = rest[nslot:2 * nslot]
        acc = rest[2 * nslot]
        dsem = rest[2 * nslot + 1:3 * nslot + 1]
        gsem = rest[3 * nslot + 1:4 * nslot + 1]
        c = lax.axis_index("c")
        s = lax.axis_index("s")

        def start_block(b, t):
            ebase = (s * bpt + b) * EB
            pltpu.async_copy(dst_hbm.at[pl.ds(ebase, EB)], didx[t], dsem[t])
            pltpu.async_copy(x_hbm.at[sall.at[pl.ds(b * EB, EB)]],
                             rows[t], gsem[t])

        def wait_block(t):
            pltpu.make_async_copy(
                dst_hbm.at[pl.ds(0, EB)], didx[t], dsem[t]).wait()
            pltpu.make_async_copy(
                x_hbm.at[sall.at[pl.ds(0, EB)]], rows[t], gsem[t]).wait()

        for r in range(rounds):
            chunk = c * rounds + r
            pltpu.sync_copy(z_hbm, acc.at[pl.ds(s * ZPT, ZPT)])
            pltpu.sync_copy(
                srcs_hbm.at[pl.ds(chunk * ep + s * bpt * EB, bpt * EB)], sall)
            plsc.subcore_barrier()

            for t in range(nslot):
                start_block(t, t)

            def body(j, _):
                for t in range(nslot):
                    b = j * nslot + t

                    @pl.when(b < bpt)
                    def _():
                        wait_block(t)
                        pltpu.sync_copy(rows[t], acc.at[didx[t]], add=True)

                        @pl.when(b + nslot < bpt)
                        def _():
                            start_block(b + nslot, t)

                return 0

            lax.fori_loop(0, -(-bpt // nslot), body, 0)
            plsc.subcore_barrier()
            pltpu.sync_copy(acc.at[pl.ds(s * ZPT, ZPT)],
                            out_hbm.at[pl.ds(chunk * ACC_ROWS + s * ZPT, ZPT)])
            plsc.subcore_barrier()

    return k(x_flat, srcs_flat, dst_e, zrows)


def _mlp(x_ch, agg_ch, wa, ba, wb, bb):
    nc, n, _ = x_ch.shape
    dim = wb.shape[1]
    nco = dim // CH
    rb = 1000
    grid = (n // rb,)

    def body(x_ref, agg_ref, wa_ref, ba_ref, wb_ref, bb_ref, out_ref, cs_ref):
        i = pl.program_id(0)
        acc = jnp.zeros((rb, dim), jnp.float32)
        for c in range(nc):
            z = x_ref[c] + agg_ref[c]
            acc = acc + jnp.dot(z, wa_ref[c * CH:(c + 1) * CH, :],
                                preferred_element_type=jnp.float32)
        h1 = jnp.maximum(acc + ba_ref[...], 0.0)
        out = jnp.dot(h1, wb_ref[...], preferred_element_type=jnp.float32)
        out = jnp.maximum(out + bb_ref[...], 0.0)
        for c in range(nco):
            out_ref[c] = out[:, c * CH:(c + 1) * CH]

        @pl.when(i == 0)
        def _():
            cs_ref[...] = jnp.zeros_like(cs_ref)

        cs_ref[...] += jnp.sum(out, axis=0, keepdims=True)

    return pl.pallas_call(
        body,
        grid=grid,
        in_specs=[
            pl.BlockSpec((nc, rb, CH), lambda i: (0, i, 0)),
            pl.BlockSpec((nc, rb, CH), lambda i: (0, i, 0)),
            pl.BlockSpec(wa.shape, lambda i: (0, 0)),
            pl.BlockSpec(ba.shape, lambda i: (0, 0)),
            pl.BlockSpec(wb.shape, lambda i: (0, 0)),
            pl.BlockSpec(bb.shape, lambda i: (0, 0)),
        ],
        out_specs=[
            pl.BlockSpec((nco, rb, CH), lambda i: (0, i, 0)),
            pl.BlockSpec((1, dim), lambda i: (0, 0)),
        ],
        out_shape=[
            jax.ShapeDtypeStruct((nco, n, CH), jnp.float32),
            jax.ShapeDtypeStruct((1, dim), jnp.float32),
        ],
        compiler_params=pltpu.CompilerParams(
            dimension_semantics=("arbitrary",)),
    )(x_ch, agg_ch, wa, ba, wb, bb)


def _readout(cs_list, lt_list):
    def body(*refs):
        out_ref = refs[-1]
        tot = jnp.zeros((1, 1), jnp.float32)
        for i in range(5):
            cs = refs[i][...]
            lt = refs[5 + i][...]
            tot = tot + jnp.sum(cs * lt, axis=1, keepdims=True)
        tot = tot * (1.0 / N)
        out_ref[...] = 1.0 / (1.0 + jnp.exp(-tot))

    return pl.pallas_call(
        body,
        out_shape=jax.ShapeDtypeStruct((1, 1), jnp.float32),
    )(*cs_list, *lt_list)


def kernel(h, edge_index, W1a, b1a, W1b, b1b, W2a, b2a, W2b, b2b, W3a, b3a,
           W3b, b3b, W4a, b4a, W4b, b4b, W5a, b5a, W5b, b5b, l1, l2, l3, l4,
           l5):
    n, f_in = h.shape
    e = edge_index.shape[1]
    src = edge_index[0].astype(jnp.int32)
    dst = edge_index[1].astype(jnp.int32)
    bpt, ep = _pad_edges(e)
    pad = ep - e
    src_p = jnp.concatenate([src, jnp.zeros((pad,), jnp.int32)])
    dst_p = jnp.concatenate([dst, jnp.full((pad,), DUMMY_ROW, jnp.int32)])
    srcs = {
        ncv: jnp.concatenate([src_p + c * N for c in range(ncv)])
        for ncv in (f_in // CH, 4)
    }
    zrows = jnp.zeros((ZPT, CH), jnp.float32)

    params = [(W1a, b1a, W1b, b1b), (W2a, b2a, W2b, b2b),
              (W3a, b3a, W3b, b3b), (W4a, b4a, W4b, b4b),
              (W5a, b5a, W5b, b5b)]

    x = h.reshape(n, f_in // CH, CH).transpose(1, 0, 2)
    cs_list = []
    for wa, ba, wb, bb in params:
        nc = x.shape[0]
        agg_flat = _segsum(x.reshape(nc * N, CH), srcs[nc], dst_p, zrows,
                           nc, bpt, ep)
        agg = agg_flat.reshape(nc, ACC_ROWS, CH)
        x, cs = _mlp(x, agg, wa, ba.reshape(1, -1), wb, bb.reshape(1, -1))
        cs_list.append(cs)

    lt_list = [l.reshape(1, -1) for l in (l1, l2, l3, l4, l5)]
    return _readout(cs_list, lt_list)


# speedup vs baseline: 4.7902x; 1.1914x over previous
import functools

import jax
import jax.numpy as jnp
from jax import lax
from jax.experimental import pallas as pl
from jax.experimental.pallas import tpu as pltpu
from jax.experimental.pallas import tpu_sc as plsc

N = 10000
CH = 128
EB = 112
NS = 16
NCORES = 2
ACC_ROWS = 10240
ZPT = ACC_ROWS // NS
DUMMY_ROW = N


def _pad_edges(e):
    bpt = -(-e // (NS * EB))
    return bpt, NS * bpt * EB


def _segsum(x_flat, srcs_flat, dst_e, zrows, nc, bpt, ep):
    rounds = nc // NCORES
    ki = 6
    kr = 3
    mesh = plsc.VectorSubcoreMesh(
        core_axis_name="c", subcore_axis_name="s",
        num_cores=NCORES, num_subcores=NS)

    scratch = [pltpu.VMEM((EB,), jnp.int32) for _ in range(2 * ki)]
    scratch += [pltpu.VMEM((EB, CH), jnp.float32) for _ in range(kr)]
    scratch += [pltpu.VMEM_SHARED((ACC_ROWS, CH), jnp.float32)]
    scratch += [pltpu.SemaphoreType.DMA for _ in range(2 * ki + 2 * kr)]

    @functools.partial(
        pl.kernel,
        out_type=jax.ShapeDtypeStruct((nc * ACC_ROWS, CH), jnp.float32),
        mesh=mesh,
        scratch_types=scratch,
    )
    def k(x_hbm, srcs_hbm, dst_hbm, z_hbm, out_hbm, *rest):
        sidx = rest[0:ki]
        didx = rest[ki:2 * ki]
        rows = rest[2 * ki:2 * ki + kr]
        acc = rest[2 * ki + kr]
        base = 2 * ki + kr + 1
        isem_s = rest[base:base + ki]
        isem_d = rest[base + ki:base + 2 * ki]
        gsem = rest[base + 2 * ki:base + 2 * ki + kr]
        ssem = rest[base + 2 * ki + kr:base + 2 * ki + 2 * kr]
        c = lax.axis_index("c")
        s = lax.axis_index("s")

        for r in range(rounds):
            chunk = c * rounds + r

            def load_idx(b, u):
                ebase = (s * bpt + b) * EB
                pltpu.async_copy(srcs_hbm.at[pl.ds(chunk * ep + ebase, EB)],
                                 sidx[u], isem_s[u])
                pltpu.async_copy(dst_hbm.at[pl.ds(ebase, EB)], didx[u],
                                 isem_d[u])

            def start_gather(u, t):
                pltpu.make_async_copy(
                    dst_hbm.at[pl.ds(0, EB)], sidx[u], isem_s[u]).wait()
                pltpu.async_copy(x_hbm.at[sidx[u]], rows[t], gsem[t])

            def wait_scatter(u, t):
                pltpu.make_async_copy(rows[t], acc.at[didx[u]],
                                      ssem[t]).wait()

            def start_scatter(u, t):
                pltpu.make_async_copy(
                    x_hbm.at[sidx[u]], rows[t], gsem[t]).wait()
                pltpu.make_async_copy(
                    dst_hbm.at[pl.ds(0, EB)], didx[u], isem_d[u]).wait()
                pltpu.async_copy(rows[t], acc.at[didx[u]], ssem[t], add=True)

            pltpu.sync_copy(z_hbm, acc.at[pl.ds(s * ZPT, ZPT)])
            plsc.subcore_barrier()

            for b in range(ki - 1):
                load_idx(b, b)
            for b in range(2):
                start_gather(b % ki, b % kr)

            def body(j, _):
                for u in range(ki):
                    b = j * ki + u

                    @pl.when((b >= 1) & (b <= bpt))
                    def _():
                        wait_scatter((u + ki - 1) % ki, (u + kr - 1) % kr)

                    @pl.when(b + ki - 1 < bpt)
                    def _():
                        load_idx(b + ki - 1, (u + ki - 1) % ki)

                    @pl.when(b + 2 < bpt)
                    def _():
                        start_gather((u + 2) % ki, (u + 2) % kr)

                    @pl.when(b < bpt)
                    def _():
                        start_scatter(u, u % kr)

                return 0

            lax.fori_loop(0, -(-bpt // ki), body, 0)
            pltpu.make_async_copy(
                rows[(bpt - 1) % kr],
                acc.at[didx[(bpt - 1) % ki]],
                ssem[(bpt - 1) % kr]).wait()
            plsc.subcore_barrier()
            pltpu.sync_copy(acc.at[pl.ds(s * ZPT, ZPT)],
                            out_hbm.at[pl.ds(chunk * ACC_ROWS + s * ZPT, ZPT)])
            plsc.subcore_barrier()

    return k(x_flat, srcs_flat, dst_e, zrows)


def _mlp(x_ch, agg_ch, wa, ba, wb, bb):
    nc, n, _ = x_ch.shape
    dim = wb.shape[1]
    nco = dim // CH
    rb = 1000
    grid = (n // rb,)

    def body(x_ref, agg_ref, wa_ref, ba_ref, wb_ref, bb_ref, out_ref, cs_ref):
        i = pl.program_id(0)
        acc = jnp.zeros((rb, dim), jnp.float32)
        for c in range(nc):
            z = x_ref[c] + agg_ref[c]
            acc = acc + jnp.dot(z, wa_ref[c * CH:(c + 1) * CH, :],
                                preferred_element_type=jnp.float32)
        h1 = jnp.maximum(acc + ba_ref[...], 0.0)
        out = jnp.dot(h1, wb_ref[...], preferred_element_type=jnp.float32)
        out = jnp.maximum(out + bb_ref[...], 0.0)
        for c in range(nco):
            out_ref[c] = out[:, c * CH:(c + 1) * CH]

        @pl.when(i == 0)
        def _():
            cs_ref[...] = jnp.zeros_like(cs_ref)

        cs_ref[...] += jnp.sum(out, axis=0, keepdims=True)

    return pl.pallas_call(
        body,
        grid=grid,
        in_specs=[
            pl.BlockSpec((nc, rb, CH), lambda i: (0, i, 0)),
            pl.BlockSpec((nc, rb, CH), lambda i: (0, i, 0)),
            pl.BlockSpec(wa.shape, lambda i: (0, 0)),
            pl.BlockSpec(ba.shape, lambda i: (0, 0)),
            pl.BlockSpec(wb.shape, lambda i: (0, 0)),
            pl.BlockSpec(bb.shape, lambda i: (0, 0)),
        ],
        out_specs=[
            pl.BlockSpec((nco, rb, CH), lambda i: (0, i, 0)),
            pl.BlockSpec((1, dim), lambda i: (0, 0)),
        ],
        out_shape=[
            jax.ShapeDtypeStruct((nco, n, CH), jnp.float32),
            jax.ShapeDtypeStruct((1, dim), jnp.float32),
        ],
        compiler_params=pltpu.CompilerParams(
            dimension_semantics=("arbitrary",)),
    )(x_ch, agg_ch, wa, ba, wb, bb)


def _readout(cs_list, lt_list):
    def body(*refs):
        out_ref = refs[-1]
        tot = jnp.zeros((1, 1), jnp.float32)
        for i in range(5):
            cs = refs[i][...]
            lt = refs[5 + i][...]
            tot = tot + jnp.sum(cs * lt, axis=1, keepdims=True)
        tot = tot * (1.0 / N)
        out_ref[...] = 1.0 / (1.0 + jnp.exp(-tot))

    return pl.pallas_call(
        body,
        out_shape=jax.ShapeDtypeStruct((1, 1), jnp.float32),
    )(*cs_list, *lt_list)


def kernel(h, edge_index, W1a, b1a, W1b, b1b, W2a, b2a, W2b, b2b, W3a, b3a,
           W3b, b3b, W4a, b4a, W4b, b4b, W5a, b5a, W5b, b5b, l1, l2, l3, l4,
           l5):
    n, f_in = h.shape
    e = edge_index.shape[1]
    src = edge_index[0].astype(jnp.int32)
    dst = edge_index[1].astype(jnp.int32)
    bpt, ep = _pad_edges(e)
    pad = ep - e
    src_p = jnp.concatenate([src, jnp.zeros((pad,), jnp.int32)])
    dst_p = jnp.concatenate([dst, jnp.full((pad,), DUMMY_ROW, jnp.int32)])
    srcs = {
        ncv: jnp.concatenate([src_p + c * N for c in range(ncv)])
        for ncv in (f_in // CH, 4)
    }
    zrows = jnp.zeros((ZPT, CH), jnp.float32)

    params = [(W1a, b1a, W1b, b1b), (W2a, b2a, W2b, b2b),
              (W3a, b3a, W3b, b3b), (W4a, b4a, W4b, b4b),
              (W5a, b5a, W5b, b5b)]

    x = h.reshape(n, f_in // CH, CH).transpose(1, 0, 2)
    cs_list = []
    for wa, ba, wb, bb in params:
        nc = x.shape[0]
        agg_flat = _segsum(x.reshape(nc * N, CH), srcs[nc], dst_p, zrows,
                           nc, bpt, ep)
        agg = agg_flat.reshape(nc, ACC_ROWS, CH)
        x, cs = _mlp(x, agg, wa, ba.reshape(1, -1), wb, bb.reshape(1, -1))
        cs_list.append(cs)

    lt_list = [l.reshape(1, -1) for l in (l1, l2, l3, l4, l5)]
    return _readout(cs_list, lt_list)


# speedup vs baseline: 6.0472x; 1.2624x over previous
import functools

import jax
import jax.numpy as jnp
from jax import lax
from jax.experimental import pallas as pl
from jax.experimental.pallas import tpu as pltpu
from jax.experimental.pallas import tpu_sc as plsc

N = 10000
CH = 128
EB = 88
NS = 16
NCORES = 2
ACC_ROWS = 10240
ZPT = ACC_ROWS // NS
DUMMY_ROW = N


def _pad_edges(e):
    bpt = -(-e // (NS * EB))
    return bpt, NS * bpt * EB


def _segsum(x_flat, srcs_flat, dst_e, zrows, nc, bpt, ep):
    rounds = nc // NCORES
    ki = 8
    kr = 4
    mesh = plsc.VectorSubcoreMesh(
        core_axis_name="c", subcore_axis_name="s",
        num_cores=NCORES, num_subcores=NS)

    scratch = [pltpu.VMEM((EB,), jnp.int32) for _ in range(2 * ki)]
    scratch += [pltpu.VMEM((EB, CH), jnp.float32) for _ in range(kr)]
    scratch += [pltpu.VMEM_SHARED((ACC_ROWS, CH), jnp.float32)]
    scratch += [pltpu.SemaphoreType.DMA for _ in range(2 * ki + 2 * kr)]

    @functools.partial(
        pl.kernel,
        out_type=jax.ShapeDtypeStruct((nc * ACC_ROWS, CH), jnp.float32),
        mesh=mesh,
        scratch_types=scratch,
    )
    def k(x_hbm, srcs_hbm, dst_hbm, z_hbm, out_hbm, *rest):
        sidx = rest[0:ki]
        didx = rest[ki:2 * ki]
        rows = rest[2 * ki:2 * ki + kr]
        acc = rest[2 * ki + kr]
        base = 2 * ki + kr + 1
        isem_s = rest[base:base + ki]
        isem_d = rest[base + ki:base + 2 * ki]
        gsem = rest[base + 2 * ki:base + 2 * ki + kr]
        ssem = rest[base + 2 * ki + kr:base + 2 * ki + 2 * kr]
        c = lax.axis_index("c")
        s = lax.axis_index("s")

        for r in range(rounds):
            chunk = c * rounds + r

            def load_sidx(b, u):
                ebase = (s * bpt + b) * EB
                pltpu.async_copy(srcs_hbm.at[pl.ds(chunk * ep + ebase, EB)],
                                 sidx[u], isem_s[u])

            def load_didx(b, u):
                ebase = (s * bpt + b) * EB
                pltpu.async_copy(dst_hbm.at[pl.ds(ebase, EB)], didx[u],
                                 isem_d[u])

            def start_gather(u, t):
                pltpu.make_async_copy(
                    dst_hbm.at[pl.ds(0, EB)], sidx[u], isem_s[u]).wait()
                pltpu.async_copy(x_hbm.at[sidx[u]], rows[t], gsem[t])

            def retire_scatter(u, t):
                pltpu.make_async_copy(rows[t], acc.at[didx[u]],
                                      ssem[t]).wait()

            def start_scatter(u, t):
                pltpu.make_async_copy(
                    x_hbm.at[sidx[u]], rows[t], gsem[t]).wait()
                pltpu.make_async_copy(
                    dst_hbm.at[pl.ds(0, EB)], didx[u], isem_d[u]).wait()
                pltpu.async_copy(rows[t], acc.at[didx[u]], ssem[t], add=True)

            pltpu.sync_copy(z_hbm, acc.at[pl.ds(s * ZPT, ZPT)])
            plsc.subcore_barrier()

            for b in range(min(ki - 1, bpt)):
                load_sidx(b, b)
            for b in range(min(ki - 2, bpt)):
                load_didx(b, b)
            for b in range(min(2, bpt)):
                start_gather(b % ki, b % kr)

            def body(j, _):
                for u in range(ki):
                    b = j * ki + u

                    @pl.when((b >= 2) & (b <= bpt + 1))
                    def _():
                        retire_scatter((u + ki - 2) % ki, (u + kr - 2) % kr)

                    @pl.when(b + ki - 1 < bpt)
                    def _():
                        load_sidx(b + ki - 1, (u + ki - 1) % ki)

                    @pl.when(b + ki - 2 < bpt)
                    def _():
                        load_didx(b + ki - 2, (u + ki - 2) % ki)

                    @pl.when(b + 2 < bpt)
                    def _():
                        start_gather((u + 2) % ki, (u + 2) % kr)

                    @pl.when(b < bpt)
                    def _():
                        start_scatter(u, u % kr)

                return 0

            nturn = -(-bpt // ki)
            lax.fori_loop(0, nturn, body, 0)
            for x in (bpt - 2, bpt - 1):
                if x >= 0 and x + 2 > ki * nturn - 1:
                    retire_scatter(x % ki, x % kr)
            plsc.subcore_barrier()
            pltpu.sync_copy(acc.at[pl.ds(s * ZPT, ZPT)],
                            out_hbm.at[pl.ds(chunk * ACC_ROWS + s * ZPT, ZPT)])
            plsc.subcore_barrier()

    return k(x_flat, srcs_flat, dst_e, zrows)


def _mlp(x_ch, agg_ch, wa, ba, wb, bb):
    nc, n, _ = x_ch.shape
    dim = wb.shape[1]
    nco = dim // CH
    rb = 1000
    grid = (n // rb,)

    def body(x_ref, agg_ref, wa_ref, ba_ref, wb_ref, bb_ref, out_ref, cs_ref):
        i = pl.program_id(0)
        acc = jnp.zeros((rb, dim), jnp.float32)
        for c in range(nc):
            z = x_ref[c] + agg_ref[c]
            acc = acc + jnp.dot(z, wa_ref[c * CH:(c + 1) * CH, :],
                                preferred_element_type=jnp.float32)
        h1 = jnp.maximum(acc + ba_ref[...], 0.0)
        out = jnp.dot(h1, wb_ref[...], preferred_element_type=jnp.float32)
        out = jnp.maximum(out + bb_ref[...], 0.0)
        for c in range(nco):
            out_ref[c] = out[:, c * CH:(c + 1) * CH]

        @pl.when(i == 0)
        def _():
            cs_ref[...] = jnp.zeros_like(cs_ref)

        cs_ref[...] += jnp.sum(out, axis=0, keepdims=True)

    return pl.pallas_call(
        body,
        grid=grid,
        in_specs=[
            pl.BlockSpec((nc, rb, CH), lambda i: (0, i, 0)),
            pl.BlockSpec((nc, rb, CH), lambda i: (0, i, 0)),
            pl.BlockSpec(wa.shape, lambda i: (0, 0)),
            pl.BlockSpec(ba.shape, lambda i: (0, 0)),
            pl.BlockSpec(wb.shape, lambda i: (0, 0)),
            pl.BlockSpec(bb.shape, lambda i: (0, 0)),
        ],
        out_specs=[
            pl.BlockSpec((nco, rb, CH), lambda i: (0, i, 0)),
            pl.BlockSpec((1, dim), lambda i: (0, 0)),
        ],
        out_shape=[
            jax.ShapeDtypeStruct((nco, n, CH), jnp.float32),
            jax.ShapeDtypeStruct((1, dim), jnp.float32),
        ],
        compiler_params=pltpu.CompilerParams(
            dimension_semantics=("arbitrary",)),
    )(x_ch, agg_ch, wa, ba, wb, bb)


def _readout(cs_list, lt_list):
    def body(*refs):
        out_ref = refs[-1]
        tot = jnp.zeros((1, 1), jnp.float32)
        for i in range(5):
            cs = refs[i][...]
            lt = refs[5 + i][...]
            tot = tot + jnp.sum(cs * lt, axis=1, keepdims=True)
        tot = tot * (1.0 / N)
        out_ref[...] = 1.0 / (1.0 + jnp.exp(-tot))

    return pl.pallas_call(
        body,
        out_shape=jax.ShapeDtypeStruct((1, 1), jnp.float32),
    )(*cs_list, *lt_list)


def kernel(h, edge_index, W1a, b1a, W1b, b1b, W2a, b2a, W2b, b2b, W3a, b3a,
           W3b, b3b, W4a, b4a, W4b, b4b, W5a, b5a, W5b, b5b, l1, l2, l3, l4,
           l5):
    n, f_in = h.shape
    e = edge_index.shape[1]
    src = edge_index[0].astype(jnp.int32)
    dst = edge_index[1].astype(jnp.int32)
    bpt, ep = _pad_edges(e)
    pad = ep - e
    src_p = jnp.concatenate([src, jnp.zeros((pad,), jnp.int32)])
    dst_p = jnp.concatenate([dst, jnp.full((pad,), DUMMY_ROW, jnp.int32)])
    srcs = {
        ncv: jnp.concatenate([src_p + c * N for c in range(ncv)])
        for ncv in (f_in // CH, 4)
    }
    zrows = jnp.zeros((ZPT, CH), jnp.float32)

    params = [(W1a, b1a, W1b, b1b), (W2a, b2a, W2b, b2b),
              (W3a, b3a, W3b, b3b), (W4a, b4a, W4b, b4b),
              (W5a, b5a, W5b, b5b)]

    x = h.reshape(n, f_in // CH, CH).transpose(1, 0, 2)
    cs_list = []
    for wa, ba, wb, bb in params:
        nc = x.shape[0]
        agg_flat = _segsum(x.reshape(nc * N, CH), srcs[nc], dst_p, zrows,
                           nc, bpt, ep)
        agg = agg_flat.reshape(nc, ACC_ROWS, CH)
        x, cs = _mlp(x, agg, wa, ba.reshape(1, -1), wb, bb.reshape(1, -1))
        cs_list.append(cs)

    lt_list = [l.reshape(1, -1) for l in (l1, l2, l3, l4, l5)]
    return _readout(cs_list, lt_list)


# speedup vs baseline: 7.3317x; 1.2124x over previous
import functools

import jax
import jax.numpy as jnp
from jax import lax
from jax.experimental import pallas as pl
from jax.experimental.pallas import tpu as pltpu
from jax.experimental.pallas import tpu_sc as plsc

N = 10000
CH = 128
EB = 72
NS = 16
NCORES = 2
ACC_ROWS = 10240
ZPT = ACC_ROWS // NS
DUMMY_ROW = N


def _pad_edges(e):
    bpt = -(-e // (NS * EB))
    return bpt, NS * bpt * EB


def _segsum(x_flat, srcs_flat, dst_e, zrows, nc, bpt, ep):
    rounds = nc // NCORES
    ki = 10
    kr = 5
    mesh = plsc.VectorSubcoreMesh(
        core_axis_name="c", subcore_axis_name="s",
        num_cores=NCORES, num_subcores=NS)

    scratch = [pltpu.VMEM((EB,), jnp.int32) for _ in range(2 * ki)]
    scratch += [pltpu.VMEM((EB, CH), jnp.float32) for _ in range(kr)]
    scratch += [pltpu.VMEM_SHARED((ACC_ROWS, CH), jnp.float32)]
    scratch += [pltpu.SemaphoreType.DMA for _ in range(2 * ki + 2 * kr)]

    @functools.partial(
        pl.kernel,
        out_type=jax.ShapeDtypeStruct((nc * ACC_ROWS, CH), jnp.float32),
        mesh=mesh,
        scratch_types=scratch,
    )
    def k(x_hbm, srcs_hbm, dst_hbm, z_hbm, out_hbm, *rest):
        sidx = rest[0:ki]
        didx = rest[ki:2 * ki]
        rows = rest[2 * ki:2 * ki + kr]
        acc = rest[2 * ki + kr]
        base = 2 * ki + kr + 1
        isem_s = rest[base:base + ki]
        isem_d = rest[base + ki:base + 2 * ki]
        gsem = rest[base + 2 * ki:base + 2 * ki + kr]
        ssem = rest[base + 2 * ki + kr:base + 2 * ki + 2 * kr]
        c = lax.axis_index("c")
        s = lax.axis_index("s")

        for r in range(rounds):
            chunk = c * rounds + r

            def load_sidx(b, u):
                ebase = (s * bpt + b) * EB
                pltpu.async_copy(srcs_hbm.at[pl.ds(chunk * ep + ebase, EB)],
                                 sidx[u], isem_s[u])

            def load_didx(b, u):
                ebase = (s * bpt + b) * EB
                pltpu.async_copy(dst_hbm.at[pl.ds(ebase, EB)], didx[u],
                                 isem_d[u])

            def start_gather(u, t):
                pltpu.make_async_copy(
                    dst_hbm.at[pl.ds(0, EB)], sidx[u], isem_s[u]).wait()
                pltpu.async_copy(x_hbm.at[sidx[u]], rows[t], gsem[t])

            def retire_scatter(u, t):
                pltpu.make_async_copy(rows[t], acc.at[didx[u]],
                                      ssem[t]).wait()

            def start_scatter(u, t):
                pltpu.make_async_copy(
                    x_hbm.at[sidx[u]], rows[t], gsem[t]).wait()
                pltpu.make_async_copy(
                    dst_hbm.at[pl.ds(0, EB)], didx[u], isem_d[u]).wait()
                pltpu.async_copy(rows[t], acc.at[didx[u]], ssem[t], add=True)

            pltpu.sync_copy(z_hbm, acc.at[pl.ds(s * ZPT, ZPT)])
            plsc.subcore_barrier()

            for b in range(min(ki - 1, bpt)):
                load_sidx(b, b)
            for b in range(min(ki - 2, bpt)):
                load_didx(b, b)
            for b in range(min(3, bpt)):
                start_gather(b % ki, b % kr)

            def body(j, _):
                for u in range(ki):
                    b = j * ki + u

                    @pl.when((b >= 2) & (b <= bpt + 1))
                    def _():
                        retire_scatter((u + ki - 2) % ki, (u + kr - 2) % kr)

                    @pl.when(b + ki - 1 < bpt)
                    def _():
                        load_sidx(b + ki - 1, (u + ki - 1) % ki)

                    @pl.when(b + ki - 2 < bpt)
                    def _():
                        load_didx(b + ki - 2, (u + ki - 2) % ki)

                    @pl.when(b + 3 < bpt)
                    def _():
                        start_gather((u + 3) % ki, (u + 3) % kr)

                    @pl.when(b < bpt)
                    def _():
                        start_scatter(u, u % kr)

                return 0

            nturn = -(-bpt // ki)
            lax.fori_loop(0, nturn, body, 0)
            for x in (bpt - 2, bpt - 1):
                if x >= 0 and x + 2 > ki * nturn - 1:
                    retire_scatter(x % ki, x % kr)
            plsc.subcore_barrier()
            pltpu.sync_copy(acc.at[pl.ds(s * ZPT, ZPT)],
                            out_hbm.at[pl.ds(chunk * ACC_ROWS + s * ZPT, ZPT)])

    return k(x_flat, srcs_flat, dst_e, zrows)


def _mlp(x_ch, agg_ch, wa, ba, wb, bb):
    nc, n, _ = x_ch.shape
    dim = wb.shape[1]
    nco = dim // CH
    rb = 1000
    grid = (n // rb,)

    def body(x_ref, agg_ref, wa_ref, ba_ref, wb_ref, bb_ref, out_ref, cs_ref):
        i = pl.program_id(0)
        acc = jnp.zeros((rb, dim), jnp.float32)
        for c in range(nc):
            z = x_ref[c] + agg_ref[c]
            acc = acc + jnp.dot(z, wa_ref[c * CH:(c + 1) * CH, :],
                                preferred_element_type=jnp.float32)
        h1 = jnp.maximum(acc + ba_ref[...], 0.0)
        out = jnp.dot(h1, wb_ref[...], preferred_element_type=jnp.float32)
        out = jnp.maximum(out + bb_ref[...], 0.0)
        for c in range(nco):
            out_ref[c] = out[:, c * CH:(c + 1) * CH]

        @pl.when(i == 0)
        def _():
            cs_ref[...] = jnp.zeros_like(cs_ref)

        cs_ref[...] += jnp.sum(out, axis=0, keepdims=True)

    return pl.pallas_call(
        body,
        grid=grid,
        in_specs=[
            pl.BlockSpec((nc, rb, CH), lambda i: (0, i, 0)),
            pl.BlockSpec((nc, rb, CH), lambda i: (0, i, 0)),
            pl.BlockSpec(wa.shape, lambda i: (0, 0)),
            pl.BlockSpec(ba.shape, lambda i: (0, 0)),
            pl.BlockSpec(wb.shape, lambda i: (0, 0)),
            pl.BlockSpec(bb.shape, lambda i: (0, 0)),
        ],
        out_specs=[
            pl.BlockSpec((nco, rb, CH), lambda i: (0, i, 0)),
            pl.BlockSpec((1, dim), lambda i: (0, 0)),
        ],
        out_shape=[
            jax.ShapeDtypeStruct((nco, n, CH), jnp.float32),
            jax.ShapeDtypeStruct((1, dim), jnp.float32),
        ],
        compiler_params=pltpu.CompilerParams(
            dimension_semantics=("arbitrary",)),
    )(x_ch, agg_ch, wa, ba, wb, bb)


def _readout(cs_list, lt_list):
    def body(*refs):
        out_ref = refs[-1]
        tot = jnp.zeros((1, 1), jnp.float32)
        for i in range(5):
            cs = refs[i][...]
            lt = refs[5 + i][...]
            tot = tot + jnp.sum(cs * lt, axis=1, keepdims=True)
        tot = tot * (1.0 / N)
        out_ref[...] = 1.0 / (1.0 + jnp.exp(-tot))

    return pl.pallas_call(
        body,
        out_shape=jax.ShapeDtypeStruct((1, 1), jnp.float32),
    )(*cs_list, *lt_list)


def kernel(h, edge_index, W1a, b1a, W1b, b1b, W2a, b2a, W2b, b2b, W3a, b3a,
           W3b, b3b, W4a, b4a, W4b, b4b, W5a, b5a, W5b, b5b, l1, l2, l3, l4,
           l5):
    n, f_in = h.shape
    e = edge_index.shape[1]
    src = edge_index[0].astype(jnp.int32)
    dst = edge_index[1].astype(jnp.int32)
    bpt, ep = _pad_edges(e)
    pad = ep - e
    src_p = jnp.concatenate([src, jnp.zeros((pad,), jnp.int32)])
    dst_p = jnp.concatenate([dst, jnp.full((pad,), DUMMY_ROW, jnp.int32)])
    srcs = {
        ncv: jnp.concatenate([src_p + c * N for c in range(ncv)])
        for ncv in (f_in // CH, 4)
    }
    zrows = jnp.zeros((ZPT, CH), jnp.float32)

    params = [(W1a, b1a, W1b, b1b), (W2a, b2a, W2b, b2b),
              (W3a, b3a, W3b, b3b), (W4a, b4a, W4b, b4b),
              (W5a, b5a, W5b, b5b)]

    x = h.reshape(n, f_in // CH, CH).transpose(1, 0, 2)
    cs_list = []
    for wa, ba, wb, bb in params:
        nc = x.shape[0]
        agg_flat = _segsum(x.reshape(nc * N, CH), srcs[nc], dst_p, zrows,
                           nc, bpt, ep)
        agg = agg_flat.reshape(nc, ACC_ROWS, CH)
        x, cs = _mlp(x, agg, wa, ba.reshape(1, -1), wb, bb.reshape(1, -1))
        cs_list.append(cs)

    lt_list = [l.reshape(1, -1) for l in (l1, l2, l3, l4, l5)]
    return _readout(cs_list, lt_list)
